# Initial kernel scaffold; baseline (speedup 1.0000x reference)
import functools

import jax
import jax.numpy as jnp
from jax import lax
from jax.experimental import pallas as pl
from jax.experimental.pallas import tpu as pltpu
from jax.experimental.pallas import tpu_sc as plsc

N_CUR = 10000
D = 128
NP = 320000
NC, NS = 2, 16
NW = NC * NS
CH = 128
K = (NP // NW + CH - 1) // CH
PTS_PER_TILE = K * CH
PADP = NW * PTS_PER_TILE
DUMMY = N_CUR
ZR = 628
ACC_ROWS = ZR * NS
CW = 16


def _sc_body(feat_hbm, idxf_hbm, seg_hbm, part_hbm, cnt_hbm,
             idxf_v, seg_v, zrows, rows, ones_v, zc, acc, cnt, gsem):
    c = lax.axis_index("c")
    s = lax.axis_index("s")
    g = c * NS + s

    pltpu.sync_copy(idxf_hbm.at[g], idxf_v)
    pltpu.sync_copy(seg_hbm.at[g], seg_v)

    lane = lax.iota(jnp.int32, 16)
    one_row = jnp.where(lane == 0, 1.0, 0.0).astype(jnp.float32)
    zero16 = jnp.zeros((16,), jnp.float32)

    def init_body(i, _):
        for d in range(D // 16):
            zrows[i, pl.ds(d * 16, 16)] = zero16
        ones_v[i] = one_row
        zc[i] = zero16
        return 0

    lax.fori_loop(0, CH, init_body, 0)

    base = s * ZR
    for k in range(ZR // CH):
        pltpu.sync_copy(zrows, acc.at[pl.ds(base + k * CH, CH)])
        pltpu.sync_copy(zc, cnt.at[pl.ds(base + k * CH, CH)])
    rem = ZR % CH
    if rem:
        off = base + (ZR // CH) * CH
        pltpu.sync_copy(zrows.at[pl.ds(0, rem)], acc.at[pl.ds(off, rem)])
        pltpu.sync_copy(zc.at[pl.ds(0, rem)], cnt.at[pl.ds(off, rem)])

    plsc.subcore_barrier()

    def chunk_body(j, _):
        pltpu.async_copy(feat_hbm.at[idxf_v.at[j]], rows, gsem).wait()
        pltpu.sync_copy(rows, acc.at[seg_v.at[j]], add=True)
        pltpu.sync_copy(ones_v, cnt.at[seg_v.at[j]], add=True)
        return 0

    lax.fori_loop(0, K, chunk_body, 0)

    plsc.subcore_barrier()

    pltpu.sync_copy(acc.at[pl.ds(base, ZR)], part_hbm.at[c, pl.ds(base, ZR)])
    pltpu.sync_copy(cnt.at[pl.ds(base, ZR)], cnt_hbm.at[c, pl.ds(base, ZR)])


_sc_call = pl.kernel(
    _sc_body,
    out_type=[
        jax.ShapeDtypeStruct((NC, ACC_ROWS, D), jnp.float32),
        jax.ShapeDtypeStruct((NC, ACC_ROWS, CW), jnp.float32),
    ],
    mesh=plsc.VectorSubcoreMesh(
        core_axis_name="c", subcore_axis_name="s",
        num_cores=NC, num_subcores=NS),
    scratch_types=[
        pltpu.VMEM((K, CH), jnp.int32),
        pltpu.VMEM((K, CH), jnp.int32),
        pltpu.VMEM((CH, D), jnp.float32),
        pltpu.VMEM((CH, D), jnp.float32),
        pltpu.VMEM((CH, CW), jnp.float32),
        pltpu.VMEM((CH, CW), jnp.float32),
        pltpu.VMEM_SHARED((ACC_ROWS, D), jnp.float32),
        pltpu.VMEM_SHARED((ACC_ROWS, CW), jnp.float32),
        pltpu.SemaphoreType.DMA,
    ],
)


def _comb_body(p_ref, c_ref, o_ref):
    total = p_ref[0] + p_ref[1]
    counts = c_ref[0, :, 0:1] + c_ref[1, :, 0:1]
    o_ref[...] = total / jnp.maximum(counts, 1.0)


_COMB_BS = 2000


def _combine(part, cnt):
    grid = N_CUR // _COMB_BS
    return pl.pallas_call(
        _comb_body,
        grid=(grid,),
        in_specs=[
            pl.BlockSpec((NC, _COMB_BS, D), lambda i: (0, i, 0)),
            pl.BlockSpec((NC, _COMB_BS, CW), lambda i: (0, i, 0)),
        ],
        out_specs=pl.BlockSpec((_COMB_BS, D), lambda i: (i, 0)),
        out_shape=jax.ShapeDtypeStruct((N_CUR, D), jnp.float32),
    )(part, cnt)


@jax.jit
def kernel(features, coors_inv_last, coors_inv, coors):
    del coors
    idxf = jnp.concatenate(
        [coors_inv_last.astype(jnp.int32),
         jnp.zeros((PADP - NP,), jnp.int32)]).reshape(NW, K, CH)
    seg = jnp.concatenate(
        [coors_inv.astype(jnp.int32),
         jnp.full((PADP - NP,), DUMMY, jnp.int32)]).reshape(NW, K, CH)
    part, cnt = _sc_call(features, idxf, seg)
    return _combine(part, cnt)


# speedup vs baseline: 6.6101x; 6.6101x over previous
import jax
import jax.numpy as jnp
from jax import lax
from jax.experimental import pallas as pl
from jax.experimental.pallas import tpu as pltpu
from jax.experimental.pallas import tpu_sc as plsc

N_CUR = 10000
D = 128
DH = D // 2
NP = 320000
NC, NS = 2, 16
CH = 128
K = (NP // NS + CH - 1) // CH
PTS_PER_TILE = K * CH
PADP = NS * PTS_PER_TILE
DUMMY = N_CUR
ZR = 632
ACC_ROWS = ZR * NS
CW = 16


def _sc_body(feat_hbm, idxf_hbm, seg_hbm, part_hbm, cnt_hbm,
             idxf_v, seg_v, zrows, rows, ones_v, zc, acc, cnt, gsem):
    c = lax.axis_index("c")
    s = lax.axis_index("s")

    pltpu.sync_copy(idxf_hbm.at[c, s], idxf_v)
    pltpu.sync_copy(seg_hbm.at[s], seg_v)

    lane = lax.iota(jnp.int32, 16)
    one_row = jnp.where(lane == 0, 1.0, 0.0).astype(jnp.float32)
    zero16 = jnp.zeros((16,), jnp.float32)

    def init_body(i, _):
        for d in range(DH // 16):
            zrows[i, pl.ds(d * 16, 16)] = zero16
        ones_v[i] = one_row
        zc[i] = zero16
        return 0

    lax.fori_loop(0, CH, init_body, 0)

    base = s * ZR
    for k in range(ZR // CH):
        pltpu.sync_copy(zrows, acc.at[pl.ds(base + k * CH, CH)])
        pltpu.sync_copy(zc, cnt.at[pl.ds(base + k * CH, CH)])
    rem = ZR % CH
    if rem:
        off = base + (ZR // CH) * CH
        pltpu.sync_copy(zrows.at[pl.ds(0, rem)], acc.at[pl.ds(off, rem)])
        pltpu.sync_copy(zc.at[pl.ds(0, rem)], cnt.at[pl.ds(off, rem)])

    plsc.subcore_barrier()

    def chunk_body(j, _):
        pltpu.async_copy(feat_hbm.at[idxf_v.at[j]], rows, gsem).wait()
        pltpu.sync_copy(rows, acc.at[seg_v.at[j]], add=True)

        @pl.when(c == 0)
        def _():
            pltpu.sync_copy(ones_v, cnt.at[seg_v.at[j]], add=True)

        return 0

    lax.fori_loop(0, K, chunk_body, 0)

    plsc.subcore_barrier()

    pltpu.sync_copy(acc.at[pl.ds(base, ZR)], part_hbm.at[c, pl.ds(base, ZR)])

    @pl.when(c == 0)
    def _():
        pltpu.sync_copy(cnt.at[pl.ds(base, ZR)], cnt_hbm.at[pl.ds(base, ZR)])


_sc_call = pl.kernel(
    _sc_body,
    out_type=[
        jax.ShapeDtypeStruct((NC, ACC_ROWS, DH), jnp.float32),
        jax.ShapeDtypeStruct((ACC_ROWS, CW), jnp.float32),
    ],
    mesh=plsc.VectorSubcoreMesh(
        core_axis_name="c", subcore_axis_name="s",
        num_cores=NC, num_subcores=NS),
    compiler_params=pltpu.CompilerParams(use_tc_tiling_on_sc=False),
    scratch_types=[
        pltpu.VMEM((K, CH), jnp.int32),
        pltpu.VMEM((K, CH), jnp.int32),
        pltpu.VMEM((CH, DH), jnp.float32),
        pltpu.VMEM((CH, DH), jnp.float32),
        pltpu.VMEM((CH, CW), jnp.float32),
        pltpu.VMEM((CH, CW), jnp.float32),
        pltpu.VMEM_SHARED((ACC_ROWS, DH), jnp.float32),
        pltpu.VMEM_SHARED((ACC_ROWS, CW), jnp.float32),
        pltpu.SemaphoreType.DMA,
    ],
)


def _comb_body(p_ref, c_ref, o_ref):
    counts = jnp.maximum(c_ref[:, 0:1], 1.0)
    o_ref[...] = jnp.concatenate([p_ref[0], p_ref[1]], axis=1) / counts


_COMB_BS = 2000


def _combine(part, cnt):
    grid = N_CUR // _COMB_BS
    return pl.pallas_call(
        _comb_body,
        grid=(grid,),
        in_specs=[
            pl.BlockSpec((NC, _COMB_BS, DH), lambda i: (0, i, 0)),
            pl.BlockSpec((_COMB_BS, CW), lambda i: (i, 0)),
        ],
        out_specs=pl.BlockSpec((_COMB_BS, D), lambda i: (i, 0)),
        out_shape=jax.ShapeDtypeStruct((N_CUR, D), jnp.float32),
    )(part, cnt)


@jax.jit
def kernel(features, coors_inv_last, coors_inv, coors):
    del coors
    feat_h = features.reshape(-1, DH)
    idxf_base = jnp.concatenate(
        [coors_inv_last.astype(jnp.int32),
         jnp.zeros((PADP - NP,), jnp.int32)])
    idxf = (idxf_base[None, :] * 2
            + jnp.arange(NC, dtype=jnp.int32)[:, None]).reshape(NC, NS, K, CH)
    seg = jnp.concatenate(
        [coors_inv.astype(jnp.int32),
         jnp.full((PADP - NP,), DUMMY, jnp.int32)]).reshape(NS, K, CH)
    part, cnt = _sc_call(feat_h, idxf, seg)
    return _combine(part, cnt)


# speedup vs baseline: 9.1347x; 1.3819x over previous
import jax
import jax.numpy as jnp
from jax import lax
from jax.experimental import pallas as pl
from jax.experimental.pallas import tpu as pltpu
from jax.experimental.pallas import tpu_sc as plsc

N_CUR = 10000
D = 128
DH = D // 2
NP = 320000
NC, NS = 2, 16
CH = 128
K = (NP // NS + CH - 1) // CH
PTS_PER_TILE = K * CH
PADP = NS * PTS_PER_TILE
DUMMY = N_CUR
ZR = 632
ACC_ROWS = ZR * NS
CW = 16


def _sc_body(feat_hbm, idxf_hbm, seg_hbm, part_hbm, cnt_hbm,
             idxf_v, seg_v, zrows, rows0, rows1, ones_v, zc, acc, cnt,
             gs0, gs1):
    c = lax.axis_index("c")
    s = lax.axis_index("s")

    pltpu.sync_copy(idxf_hbm.at[c, s], idxf_v)
    pltpu.sync_copy(seg_hbm.at[s], seg_v)

    lane = lax.iota(jnp.int32, 16)
    one_row = jnp.where(lane == 0, 1.0, 0.0).astype(jnp.float32)
    zero16 = jnp.zeros((16,), jnp.float32)

    def init_body(i, _):
        for d in range(DH // 16):
            zrows[i, pl.ds(d * 16, 16)] = zero16
        ones_v[i] = one_row
        zc[i] = zero16
        return 0

    lax.fori_loop(0, CH, init_body, 0)

    base = s * ZR
    for k in range(ZR // CH):
        pltpu.sync_copy(zrows, acc.at[pl.ds(base + k * CH, CH)])
        pltpu.sync_copy(zc, cnt.at[pl.ds(base + k * CH, CH)])
    rem = ZR % CH
    if rem:
        off = base + (ZR // CH) * CH
        pltpu.sync_copy(zrows.at[pl.ds(0, rem)], acc.at[pl.ds(off, rem)])
        pltpu.sync_copy(zc.at[pl.ds(0, rem)], cnt.at[pl.ds(off, rem)])

    plsc.subcore_barrier()

    def fire(j, buf, sem):
        pltpu.async_copy(feat_hbm.at[idxf_v.at[j]], buf, sem)

    def drain(j, buf, sem):
        pltpu.make_async_copy(feat_hbm.at[idxf_v.at[j]], buf, sem).wait()

    def scat(j, buf):
        pltpu.sync_copy(buf, acc.at[seg_v.at[j]], add=True)

        @pl.when(lax.rem(j, 2) == c)
        def _():
            pltpu.sync_copy(ones_v, cnt.at[seg_v.at[j]], add=True)

    assert K % 2 == 1
    fire(0, rows0, gs0)

    def pair_body(i, _):
        j0 = 2 * i
        fire(j0 + 1, rows1, gs1)
        drain(j0, rows0, gs0)
        scat(j0, rows0)
        fire(j0 + 2, rows0, gs0)
        drain(j0 + 1, rows1, gs1)
        scat(j0 + 1, rows1)
        return 0

    lax.fori_loop(0, K // 2, pair_body, 0)
    drain(K - 1, rows0, gs0)
    scat(K - 1, rows0)

    plsc.subcore_barrier()

    pltpu.sync_copy(acc.at[pl.ds(base, ZR)], part_hbm.at[c, pl.ds(base, ZR)])
    pltpu.sync_copy(cnt.at[pl.ds(base, ZR)], cnt_hbm.at[c, pl.ds(base, ZR)])


_sc_call = pl.kernel(
    _sc_body,
    out_type=[
        jax.ShapeDtypeStruct((NC, ACC_ROWS, DH), jnp.float32),
        jax.ShapeDtypeStruct((NC, ACC_ROWS, CW), jnp.float32),
    ],
    mesh=plsc.VectorSubcoreMesh(
        core_axis_name="c", subcore_axis_name="s",
        num_cores=NC, num_subcores=NS),
    compiler_params=pltpu.CompilerParams(use_tc_tiling_on_sc=False),
    scratch_types=[
        pltpu.VMEM((K, CH), jnp.int32),
        pltpu.VMEM((K, CH), jnp.int32),
        pltpu.VMEM((CH, DH), jnp.float32),
        pltpu.VMEM((CH, DH), jnp.float32),
        pltpu.VMEM((CH, DH), jnp.float32),
        pltpu.VMEM((CH, CW), jnp.float32),
        pltpu.VMEM((CH, CW), jnp.float32),
        pltpu.VMEM_SHARED((ACC_ROWS, DH), jnp.float32),
        pltpu.VMEM_SHARED((ACC_ROWS, CW), jnp.float32),
        pltpu.SemaphoreType.DMA,
        pltpu.SemaphoreType.DMA,
    ],
)


def _comb_body(p_ref, c_ref, o_ref):
    counts = jnp.maximum(c_ref[0, :, 0:1] + c_ref[1, :, 0:1], 1.0)
    o_ref[...] = jnp.concatenate([p_ref[0], p_ref[1]], axis=1) / counts


_COMB_BS = 2000


def _combine(part, cnt):
    grid = N_CUR // _COMB_BS
    return pl.pallas_call(
        _comb_body,
        grid=(grid,),
        in_specs=[
            pl.BlockSpec((NC, _COMB_BS, DH), lambda i: (0, i, 0)),
            pl.BlockSpec((NC, _COMB_BS, CW), lambda i: (0, i, 0)),
        ],
        out_specs=pl.BlockSpec((_COMB_BS, D), lambda i: (i, 0)),
        out_shape=jax.ShapeDtypeStruct((N_CUR, D), jnp.float32),
    )(part, cnt)


@jax.jit
def kernel(features, coors_inv_last, coors_inv, coors):
    del coors
    feat_h = features.reshape(-1, DH)
    idxf_base = jnp.concatenate(
        [coors_inv_last.astype(jnp.int32),
         jnp.zeros((PADP - NP,), jnp.int32)])
    idxf = (idxf_base[None, :] * 2
            + jnp.arange(NC, dtype=jnp.int32)[:, None]).reshape(NC, NS, K, CH)
    seg = jnp.concatenate(
        [coors_inv.astype(jnp.int32),
         jnp.full((PADP - NP,), DUMMY, jnp.int32)]).reshape(NS, K, CH)
    part, cnt = _sc_call(feat_h, idxf, seg)
    return _combine(part, cnt)


# speedup vs baseline: 9.3043x; 1.0186x over previous
import jax
import jax.numpy as jnp
from jax import lax
from jax.experimental import pallas as pl
from jax.experimental.pallas import tpu as pltpu
from jax.experimental.pallas import tpu_sc as plsc

N_CUR = 10000
D = 128
DH = D // 2
NP = 320000
NC, NS = 2, 16
NW = NC * NS
CH = 128
K = (NP // NS + CH - 1) // CH
PTS_PER_TILE = K * CH
PADP = NS * PTS_PER_TILE
DUMMY = N_CUR
ZR = 632
ACC_ROWS = ZR * NS


def _sc_body(feat_hbm, idxf_hbm, seg_hbm, part_hbm, cnt_hbm,
             idxf_v, seg_v, zrows, rows0, rows1, cnt_v, acc, gs0, gs1):
    c = lax.axis_index("c")
    s = lax.axis_index("s")
    g = c * NS + s

    pltpu.sync_copy(idxf_hbm.at[c, s], idxf_v)
    pltpu.sync_copy(seg_hbm.at[s], seg_v)

    zero16 = jnp.zeros((16,), jnp.float32)
    ones16 = jnp.ones((16,), jnp.float32)

    def zrows_body(i, _):
        for d in range(DH // 16):
            zrows[i, pl.ds(d * 16, 16)] = zero16
        return 0

    lax.fori_loop(0, CH, zrows_body, 0)

    def zcnt_body(i, _):
        cnt_v[pl.ds(i * 16, 16)] = zero16
        return 0

    lax.fori_loop(0, ACC_ROWS // 16, zcnt_body, 0)

    base = s * ZR
    for k in range(ZR // CH):
        pltpu.sync_copy(zrows, acc.at[pl.ds(base + k * CH, CH)])
    rem = ZR % CH
    if rem:
        off = base + (ZR // CH) * CH
        pltpu.sync_copy(zrows.at[pl.ds(0, rem)], acc.at[pl.ds(off, rem)])

    plsc.subcore_barrier()

    def fire(j, buf, sem):
        pltpu.async_copy(feat_hbm.at[idxf_v.at[j]], buf, sem)

    def drain(j, buf, sem):
        pltpu.make_async_copy(feat_hbm.at[idxf_v.at[j]], buf, sem).wait()

    def scat(j, buf):
        pltpu.sync_copy(buf, acc.at[seg_v.at[j]], add=True)

        @pl.when(lax.rem(j, 2) == c)
        def _():
            for l in range(CH // 16):
                sv = seg_v[j, pl.ds(l * 16, 16)]
                plsc.addupdate_scatter(cnt_v, [sv], ones16)

    assert K % 2 == 1
    fire(0, rows0, gs0)

    def pair_body(i, _):
        j0 = 2 * i
        fire(j0 + 1, rows1, gs1)
        drain(j0, rows0, gs0)
        scat(j0, rows0)
        fire(j0 + 2, rows0, gs0)
        drain(j0 + 1, rows1, gs1)
        scat(j0 + 1, rows1)
        return 0

    lax.fori_loop(0, K // 2, pair_body, 0)
    drain(K - 1, rows0, gs0)
    scat(K - 1, rows0)

    plsc.subcore_barrier()

    pltpu.sync_copy(acc.at[pl.ds(base, ZR)], part_hbm.at[c, pl.ds(base, ZR)])
    pltpu.sync_copy(cnt_v, cnt_hbm.at[g])


_sc_call = pl.kernel(
    _sc_body,
    out_type=[
        jax.ShapeDtypeStruct((NC, ACC_ROWS, DH), jnp.float32),
        jax.ShapeDtypeStruct((NW, ACC_ROWS), jnp.float32),
    ],
    mesh=plsc.VectorSubcoreMesh(
        core_axis_name="c", subcore_axis_name="s",
        num_cores=NC, num_subcores=NS),
    compiler_params=pltpu.CompilerParams(
        use_tc_tiling_on_sc=False, needs_layout_passes=False),
    scratch_types=[
        pltpu.VMEM((K, CH), jnp.int32),
        pltpu.VMEM((K, CH), jnp.int32),
        pltpu.VMEM((CH, DH), jnp.float32),
        pltpu.VMEM((CH, DH), jnp.float32),
        pltpu.VMEM((CH, DH), jnp.float32),
        pltpu.VMEM((ACC_ROWS,), jnp.float32),
        pltpu.VMEM_SHARED((ACC_ROWS, DH), jnp.float32),
        pltpu.SemaphoreType.DMA,
        pltpu.SemaphoreType.DMA,
    ],
)


def _comb_body(p_ref, c_ref, o_ref):
    counts = jnp.maximum(jnp.sum(c_ref[...], axis=1), 1.0)[:, None]
    o_ref[...] = jnp.concatenate([p_ref[0], p_ref[1]], axis=1) / counts


_COMB_BS = 2000


def _combine(part, cnt):
    grid = N_CUR // _COMB_BS
    return pl.pallas_call(
        _comb_body,
        grid=(grid,),
        in_specs=[
            pl.BlockSpec((NC, _COMB_BS, DH), lambda i: (0, i, 0)),
            pl.BlockSpec((_COMB_BS, NW), lambda i: (i, 0)),
        ],
        out_specs=pl.BlockSpec((_COMB_BS, D), lambda i: (i, 0)),
        out_shape=jax.ShapeDtypeStruct((N_CUR, D), jnp.float32),
    )(part, cnt)


@jax.jit
def kernel(features, coors_inv_last, coors_inv, coors):
    del coors
    feat_h = features.reshape(-1, DH)
    idxf_base = jnp.concatenate(
        [coors_inv_last.astype(jnp.int32),
         jnp.zeros((PADP - NP,), jnp.int32)])
    idxf = (idxf_base[None, :] * 2
            + jnp.arange(NC, dtype=jnp.int32)[:, None]).reshape(NC, NS, K, CH)
    seg = jnp.concatenate(
        [coors_inv.astype(jnp.int32),
         jnp.full((PADP - NP,), DUMMY, jnp.int32)]).reshape(NS, K, CH)
    part, cnt = _sc_call(feat_h, idxf, seg)
    return _combine(part, cnt.T)


# speedup vs baseline: 10.2684x; 1.1036x over previous
import jax
import jax.numpy as jnp
from jax import lax
from jax.experimental import pallas as pl
from jax.experimental.pallas import tpu as pltpu
from jax.experimental.pallas import tpu_sc as plsc

N_CUR = 10000
D = 128
DH = D // 2
NP = 320000
NC, NS = 2, 16
NW = NC * NS
CH = 128
K = (NP // NS + CH - 1) // CH
PTS_PER_TILE = K * CH
PADP = NS * PTS_PER_TILE
DUMMY = N_CUR
ZR = 632
ACC_ROWS = ZR * NS


def _sc_body(feat_hbm, idxf_hbm, seg_hbm, part_hbm, cnt_hbm,
             idxf_v, seg_v, rows0, rows1, rows2, rows3, cnt_v, acc,
             gs0, gs1, gs2, gs3):
    c = lax.axis_index("c")
    s = lax.axis_index("s")
    g = c * NS + s

    pltpu.sync_copy(idxf_hbm.at[c, s], idxf_v)
    pltpu.sync_copy(seg_hbm.at[s], seg_v)

    zero16 = jnp.zeros((16,), jnp.float32)
    ones16 = jnp.ones((16,), jnp.float32)

    def zrows_body(i, _):
        for d in range(DH // 16):
            rows0[i, pl.ds(d * 16, 16)] = zero16
        return 0

    lax.fori_loop(0, CH, zrows_body, 0)

    def zcnt_body(i, _):
        cnt_v[pl.ds(i * 16, 16)] = zero16
        return 0

    lax.fori_loop(0, ACC_ROWS // 16, zcnt_body, 0)

    base = s * ZR
    for k in range(ZR // CH):
        pltpu.sync_copy(rows0, acc.at[pl.ds(base + k * CH, CH)])
    rem = ZR % CH
    if rem:
        off = base + (ZR // CH) * CH
        pltpu.sync_copy(rows0.at[pl.ds(0, rem)], acc.at[pl.ds(off, rem)])

    plsc.subcore_barrier()

    def fire(j, buf, sem):
        pltpu.async_copy(feat_hbm.at[idxf_v.at[j]], buf, sem)

    def drain(j, buf, sem):
        pltpu.make_async_copy(feat_hbm.at[idxf_v.at[j]], buf, sem).wait()

    def scat(j, buf):
        pltpu.sync_copy(buf, acc.at[seg_v.at[j]], add=True)

        @pl.when(lax.rem(j, 2) == c)
        def _():
            for l in range(CH // 16):
                sv = seg_v[j, pl.ds(l * 16, 16)]
                plsc.addupdate_scatter(cnt_v, [sv], ones16)

    bufs = (rows0, rows1, rows2, rows3)
    sems = (gs0, gs1, gs2, gs3)
    NB = len(bufs)
    for b in range(NB):
        fire(b, bufs[b], sems[b])

    NQ = (K - 1) // NB - 1
    def quad_body(q, _):
        for b in range(NB):
            j = NB * q + b
            drain(j, bufs[b], sems[b])
            scat(j, bufs[b])
            fire(j + NB, bufs[b], sems[b])
        return 0

    lax.fori_loop(0, NQ, quad_body, 0)
    for j in range(NB * NQ, K):
        b = j % NB
        drain(j, bufs[b], sems[b])
        scat(j, bufs[b])
        if j + NB < K:
            fire(j + NB, bufs[b], sems[b])

    plsc.subcore_barrier()

    pltpu.sync_copy(acc.at[pl.ds(base, ZR)], part_hbm.at[c, pl.ds(base, ZR)])
    pltpu.sync_copy(cnt_v, cnt_hbm.at[g])


_sc_call = pl.kernel(
    _sc_body,
    out_type=[
        jax.ShapeDtypeStruct((NC, ACC_ROWS, DH), jnp.float32),
        jax.ShapeDtypeStruct((NW, ACC_ROWS), jnp.float32),
    ],
    mesh=plsc.VectorSubcoreMesh(
        core_axis_name="c", subcore_axis_name="s",
        num_cores=NC, num_subcores=NS),
    compiler_params=pltpu.CompilerParams(
        use_tc_tiling_on_sc=False, needs_layout_passes=False),
    scratch_types=[
        pltpu.VMEM((K, CH), jnp.int32),
        pltpu.VMEM((K, CH), jnp.int32),
        pltpu.VMEM((CH, DH), jnp.float32),
        pltpu.VMEM((CH, DH), jnp.float32),
        pltpu.VMEM((CH, DH), jnp.float32),
        pltpu.VMEM((CH, DH), jnp.float32),
        pltpu.VMEM((ACC_ROWS,), jnp.float32),
        pltpu.VMEM_SHARED((ACC_ROWS, DH), jnp.float32),
        pltpu.SemaphoreType.DMA,
        pltpu.SemaphoreType.DMA,
        pltpu.SemaphoreType.DMA,
        pltpu.SemaphoreType.DMA,
    ],
)


def _comb_body(p_ref, c_ref, o_ref):
    counts = jnp.maximum(jnp.sum(c_ref[...], axis=1), 1.0)[:, None]
    o_ref[...] = jnp.concatenate([p_ref[0], p_ref[1]], axis=1) / counts


_COMB_BS = 2000


def _combine(part, cnt):
    grid = N_CUR // _COMB_BS
    return pl.pallas_call(
        _comb_body,
        grid=(grid,),
        in_specs=[
            pl.BlockSpec((NC, _COMB_BS, DH), lambda i: (0, i, 0)),
            pl.BlockSpec((_COMB_BS, NW), lambda i: (i, 0)),
        ],
        out_specs=pl.BlockSpec((_COMB_BS, D), lambda i: (i, 0)),
        out_shape=jax.ShapeDtypeStruct((N_CUR, D), jnp.float32),
    )(part, cnt)


@jax.jit
def kernel(features, coors_inv_last, coors_inv, coors):
    del coors
    feat_h = features.reshape(-1, DH)
    idxf_base = jnp.concatenate(
        [coors_inv_last.astype(jnp.int32),
         jnp.zeros((PADP - NP,), jnp.int32)])
    idxf = (idxf_base[None, :] * 2
            + jnp.arange(NC, dtype=jnp.int32)[:, None]).reshape(NC, NS, K, CH)
    seg = jnp.concatenate(
        [coors_inv.astype(jnp.int32),
         jnp.full((PADP - NP,), DUMMY, jnp.int32)]).reshape(NS, K, CH)
    part, cnt = _sc_call(feat_h, idxf, seg)
    return _combine(part, cnt.T)


# speedup vs baseline: 10.4753x; 1.0201x over previous
import jax
import jax.numpy as jnp
from jax import lax
from jax.experimental import pallas as pl
from jax.experimental.pallas import tpu as pltpu
from jax.experimental.pallas import tpu_sc as plsc

N_CUR = 10000
D = 128
DH = D // 2
NP = 320000
NC, NS = 2, 16
NW = NC * NS
CH = 128
K = (NP // NS + CH - 1) // CH
PTS_PER_TILE = K * CH
PADP = NS * PTS_PER_TILE
DUMMY = N_CUR
ZR = 632
ACC_ROWS = ZR * NS


def _sc_body(feat_hbm, idxf_hbm, seg_hbm, part_hbm, cnt_hbm,
             idxf_v, seg_v, rows0, rows1, rows2, rows3, cnt_v, acc,
             gs0, gs1, gs2, gs3, ss0, ss1, ss2, ss3):
    c = lax.axis_index("c")
    s = lax.axis_index("s")
    g = c * NS + s

    pltpu.sync_copy(idxf_hbm.at[c, s], idxf_v)
    pltpu.sync_copy(seg_hbm.at[s], seg_v)

    zero16 = jnp.zeros((16,), jnp.float32)
    ones16 = jnp.ones((16,), jnp.float32)

    def zrows_body(i, _):
        for d in range(DH // 16):
            rows0[i, pl.ds(d * 16, 16)] = zero16
        return 0

    lax.fori_loop(0, CH, zrows_body, 0)

    def zcnt_body(i, _):
        cnt_v[pl.ds(i * 16, 16)] = zero16
        return 0

    lax.fori_loop(0, ACC_ROWS // 16, zcnt_body, 0)

    base = s * ZR
    for k in range(ZR // CH):
        pltpu.sync_copy(rows0, acc.at[pl.ds(base + k * CH, CH)])
    rem = ZR % CH
    if rem:
        off = base + (ZR // CH) * CH
        pltpu.sync_copy(rows0.at[pl.ds(0, rem)], acc.at[pl.ds(off, rem)])

    plsc.subcore_barrier()

    def fire(j, buf, sem):
        pltpu.async_copy(feat_hbm.at[idxf_v.at[j]], buf, sem)

    def drain(j, buf, sem):
        pltpu.make_async_copy(feat_hbm.at[idxf_v.at[j]], buf, sem).wait()

    def fire_s(j, buf, sem):
        pltpu.async_copy(buf, acc.at[seg_v.at[j]], sem, add=True)

    def wait_s(j, buf, sem):
        pltpu.make_async_copy(buf, acc.at[seg_v.at[j]], sem).wait()

    def count(j):
        @pl.when(lax.rem(j, 2) == c)
        def _():
            for l in range(CH // 16):
                sv = seg_v[j, pl.ds(l * 16, 16)]
                plsc.addupdate_scatter(cnt_v, [sv], ones16)

    bufs = (rows0, rows1, rows2, rows3)
    sems = (gs0, gs1, gs2, gs3)
    ssems = (ss0, ss1, ss2, ss3)
    NB = len(bufs)

    def slot(j, b, do_wait=True, fire_next=True):
        drain(j, bufs[b], sems[b])
        fire_s(j, bufs[b], ssems[b])
        count(j)
        bp = (b - 1) % NB
        if do_wait:
            wait_s(j - 1, bufs[bp], ssems[bp])
        if fire_next:
            fire(j + NB - 1, bufs[bp], sems[bp])

    for b in range(NB - 1):
        fire(b, bufs[b], sems[b])
    slot(0, 0, do_wait=False)
    slot(1, 1)
    slot(2, 2)
    slot(3, 3)

    NQ_LO, NQ_HI = 1, (K - 2 * NB + 1) // NB

    def quad_body(q, _):
        j0 = NB * q
        for b in range(NB):
            slot(j0 + b, b)
        return 0

    lax.fori_loop(NQ_LO, NQ_HI + 1, quad_body, 0)
    for j in range(NB * (NQ_HI + 1), K):
        slot(j, j % NB, fire_next=(j + NB - 1 < K))
    wait_s(K - 1, bufs[(K - 1) % NB], ssems[(K - 1) % NB])

    plsc.subcore_barrier()

    pltpu.sync_copy(acc.at[pl.ds(base, ZR)], part_hbm.at[c, pl.ds(base, ZR)])
    pltpu.sync_copy(cnt_v, cnt_hbm.at[g])


_sc_call = pl.kernel(
    _sc_body,
    out_type=[
        jax.ShapeDtypeStruct((NC, ACC_ROWS, DH), jnp.float32),
        jax.ShapeDtypeStruct((NW, ACC_ROWS), jnp.float32),
    ],
    mesh=plsc.VectorSubcoreMesh(
        core_axis_name="c", subcore_axis_name="s",
        num_cores=NC, num_subcores=NS),
    compiler_params=pltpu.CompilerParams(
        use_tc_tiling_on_sc=False, needs_layout_passes=False),
    scratch_types=[
        pltpu.VMEM((K, CH), jnp.int32),
        pltpu.VMEM((K, CH), jnp.int32),
        pltpu.VMEM((CH, DH), jnp.float32),
        pltpu.VMEM((CH, DH), jnp.float32),
        pltpu.VMEM((CH, DH), jnp.float32),
        pltpu.VMEM((CH, DH), jnp.float32),
        pltpu.VMEM((ACC_ROWS,), jnp.float32),
        pltpu.VMEM_SHARED((ACC_ROWS, DH), jnp.float32),
        pltpu.SemaphoreType.DMA,
        pltpu.SemaphoreType.DMA,
        pltpu.SemaphoreType.DMA,
        pltpu.SemaphoreType.DMA,
        pltpu.SemaphoreType.DMA,
        pltpu.SemaphoreType.DMA,
        pltpu.SemaphoreType.DMA,
        pltpu.SemaphoreType.DMA,
    ],
)


def _comb_body(p_ref, c_ref, o_ref):
    counts = jnp.maximum(jnp.sum(c_ref[...], axis=1), 1.0)[:, None]
    o_ref[...] = jnp.concatenate([p_ref[0], p_ref[1]], axis=1) / counts


_COMB_BS = 2000


def _combine(part, cnt):
    grid = N_CUR // _COMB_BS
    return pl.pallas_call(
        _comb_body,
        grid=(grid,),
        in_specs=[
            pl.BlockSpec((NC, _COMB_BS, DH), lambda i: (0, i, 0)),
            pl.BlockSpec((_COMB_BS, NW), lambda i: (i, 0)),
        ],
        out_specs=pl.BlockSpec((_COMB_BS, D), lambda i: (i, 0)),
        out_shape=jax.ShapeDtypeStruct((N_CUR, D), jnp.float32),
    )(part, cnt)


@jax.jit
def kernel(features, coors_inv_last, coors_inv, coors):
    del coors
    feat_h = features.reshape(-1, DH)
    idxf_base = jnp.concatenate(
        [coors_inv_last.astype(jnp.int32),
         jnp.zeros((PADP - NP,), jnp.int32)])
    idxf = (idxf_base[None, :] * 2
            + jnp.arange(NC, dtype=jnp.int32)[:, None]).reshape(NC, NS, K, CH)
    seg = jnp.concatenate(
        [coors_inv.astype(jnp.int32),
         jnp.full((PADP - NP,), DUMMY, jnp.int32)]).reshape(NS, K, CH)
    part, cnt = _sc_call(feat_h, idxf, seg)
    return _combine(part, cnt.T)


# speedup vs baseline: 14.3747x; 1.3723x over previous
import jax
import jax.numpy as jnp
from jax import lax
from jax.experimental import pallas as pl
from jax.experimental.pallas import tpu as pltpu
from jax.experimental.pallas import tpu_sc as plsc

N_CUR = 10000
D = 128
DH = D // 2
NP = 320000
NC, NS = 2, 16
NW = NC * NS
CH = 128
NCHUNK = NP // CH
KB = NCHUNK // NS
EXTRA = NCHUNK - KB * NS
KMAX = KB + 1
ZR = 632
ACC_ROWS = ZR * NS


def _sc_body(feat_hbm, idxf_hbm, seg_hbm, part_hbm, cnt_hbm,
             idxf_v, seg_v, rows0, rows1, rows2, rows3, cnt_v, acc,
             gs0, gs1, gs2, gs3, ss0, ss1, ss2, ss3):
    c = lax.axis_index("c")
    s = lax.axis_index("s")
    g = c * NS + s
    has_extra = s < EXTRA

    pltpu.sync_copy(idxf_hbm.at[pl.ds(s * KB, KB)], idxf_v.at[pl.ds(0, KB)])
    pltpu.sync_copy(seg_hbm.at[pl.ds(s * KB, KB)], seg_v.at[pl.ds(0, KB)])

    @pl.when(has_extra)
    def _():
        pltpu.sync_copy(idxf_hbm.at[NS * KB + s], idxf_v.at[KB])
        pltpu.sync_copy(seg_hbm.at[NS * KB + s], seg_v.at[KB])

    def xform_body(i, _):
        for d in range(CH // 16):
            v = idxf_v[i, pl.ds(d * 16, 16)]
            idxf_v[i, pl.ds(d * 16, 16)] = v * 2 + c
        return 0

    lax.fori_loop(0, KMAX, xform_body, 0)

    zero16 = jnp.zeros((16,), jnp.float32)
    ones16 = jnp.ones((16,), jnp.float32)

    def zrows_body(i, _):
        for d in range(DH // 16):
            rows0[i, pl.ds(d * 16, 16)] = zero16
        return 0

    lax.fori_loop(0, CH, zrows_body, 0)

    def zcnt_body(i, _):
        cnt_v[pl.ds(i * 16, 16)] = zero16
        return 0

    lax.fori_loop(0, ACC_ROWS // 16, zcnt_body, 0)

    base = s * ZR
    for k in range(ZR // CH):
        pltpu.sync_copy(rows0, acc.at[pl.ds(base + k * CH, CH)])
    rem = ZR % CH
    if rem:
        off = base + (ZR // CH) * CH
        pltpu.sync_copy(rows0.at[pl.ds(0, rem)], acc.at[pl.ds(off, rem)])

    plsc.subcore_barrier()

    def fire(j, buf, sem):
        pltpu.async_copy(feat_hbm.at[idxf_v.at[j]], buf, sem)

    def drain(j, buf, sem):
        pltpu.make_async_copy(feat_hbm.at[idxf_v.at[j]], buf, sem).wait()

    def fire_s(j, buf, sem):
        pltpu.async_copy(buf, acc.at[seg_v.at[j]], sem, add=True)

    def wait_s(j, buf, sem):
        pltpu.make_async_copy(buf, acc.at[seg_v.at[j]], sem).wait()

    def count(j):
        @pl.when(lax.rem(j, 2) == c)
        def _():
            for l in range(CH // 16):
                sv = seg_v[j, pl.ds(l * 16, 16)]
                plsc.addupdate_scatter(cnt_v, [sv], ones16)

    bufs = (rows0, rows1, rows2, rows3)
    sems = (gs0, gs1, gs2, gs3)
    ssems = (ss0, ss1, ss2, ss3)
    NB = len(bufs)

    def slot(j, b, do_wait=True, fire_next=True):
        drain(j, bufs[b], sems[b])
        fire_s(j, bufs[b], ssems[b])
        count(j)
        bp = (b - 1) % NB
        if do_wait:
            wait_s(j - 1, bufs[bp], ssems[bp])
        if fire_next:
            fire(j + NB - 1, bufs[bp], sems[bp])

    for b in range(NB - 1):
        fire(b, bufs[b], sems[b])
    slot(0, 0, do_wait=False)
    slot(1, 1)
    slot(2, 2)
    slot(3, 3)

    NQ_LO, NQ_HI = 1, (KB - 2 * NB + 1) // NB

    def quad_body(q, _):
        j0 = NB * q
        for b in range(NB):
            slot(j0 + b, b)
        return 0

    lax.fori_loop(NQ_LO, NQ_HI + 1, quad_body, 0)
    for j in range(NB * (NQ_HI + 1), KB):
        slot(j, j % NB, fire_next=(j + NB - 1 < KB))
    wait_s(KB - 1, bufs[(KB - 1) % NB], ssems[(KB - 1) % NB])

    @pl.when(has_extra)
    def _():
        fire(KB, rows0, gs0)
        drain(KB, rows0, gs0)
        fire_s(KB, rows0, ss0)
        count(KB)
        wait_s(KB, rows0, ss0)

    plsc.subcore_barrier()

    pltpu.sync_copy(acc.at[pl.ds(base, ZR)], part_hbm.at[c, pl.ds(base, ZR)])
    pltpu.sync_copy(cnt_v, cnt_hbm.at[g])


_sc_call = pl.kernel(
    _sc_body,
    out_type=[
        jax.ShapeDtypeStruct((NC, ACC_ROWS, DH), jnp.float32),
        jax.ShapeDtypeStruct((NW, ACC_ROWS), jnp.float32),
    ],
    mesh=plsc.VectorSubcoreMesh(
        core_axis_name="c", subcore_axis_name="s",
        num_cores=NC, num_subcores=NS),
    compiler_params=pltpu.CompilerParams(
        use_tc_tiling_on_sc=False, needs_layout_passes=False),
    scratch_types=[
        pltpu.VMEM((KMAX, CH), jnp.int32),
        pltpu.VMEM((KMAX, CH), jnp.int32),
        pltpu.VMEM((CH, DH), jnp.float32),
        pltpu.VMEM((CH, DH), jnp.float32),
        pltpu.VMEM((CH, DH), jnp.float32),
        pltpu.VMEM((CH, DH), jnp.float32),
        pltpu.VMEM((ACC_ROWS,), jnp.float32),
        pltpu.VMEM_SHARED((ACC_ROWS, DH), jnp.float32),
        pltpu.SemaphoreType.DMA,
        pltpu.SemaphoreType.DMA,
        pltpu.SemaphoreType.DMA,
        pltpu.SemaphoreType.DMA,
        pltpu.SemaphoreType.DMA,
        pltpu.SemaphoreType.DMA,
        pltpu.SemaphoreType.DMA,
        pltpu.SemaphoreType.DMA,
    ],
)


def _comb_body(p_ref, c_ref, o_ref):
    p0 = p_ref[0, :N_CUR, :]
    p1 = p_ref[1, :N_CUR, :]
    counts = jnp.sum(jnp.swapaxes(c_ref[...], 0, 1), axis=1, keepdims=True)
    counts = jnp.maximum(counts, 1.0)[:N_CUR]
    o_ref[...] = jnp.concatenate([p0, p1], axis=1) / counts


def _combine(part, cnt):
    return pl.pallas_call(
        _comb_body,
        out_shape=jax.ShapeDtypeStruct((N_CUR, D), jnp.float32),
    )(part, cnt)


@jax.jit
def kernel(features, coors_inv_last, coors_inv, coors):
    del coors
    feat_h = features.reshape(-1, DH)
    idxf = coors_inv_last.astype(jnp.int32).reshape(NCHUNK, CH)
    seg = coors_inv.astype(jnp.int32).reshape(NCHUNK, CH)
    part, cnt = _sc_call(feat_h, idxf, seg)
    return _combine(part, cnt)


# speedup vs baseline: 15.2707x; 1.0623x over previous
import jax
import jax.numpy as jnp
from jax import lax
from jax.experimental import pallas as pl
from jax.experimental.pallas import tpu as pltpu
from jax.experimental.pallas import tpu_sc as plsc

N_CUR = 10000
D = 128
DH = D // 2
NP = 320000
NC, NS = 2, 16
NW = NC * NS
CH = 128
NCHUNK = NP // CH
KB = NCHUNK // NS
EXTRA = NCHUNK - KB * NS
KMAX = KB + 1
ZR = 632
ACC_ROWS = ZR * NS


def _sc_body(feat_hbm, idxf_hbm, seg_hbm, part_hbm, cnt_hbm,
             idxf_v, seg_v, rows0, rows1, rows2, rows3, cnt_v, acc,
             gs0, gs1, gs2, gs3, ss0, ss1, ss2, ss3):
    c = lax.axis_index("c")
    s = lax.axis_index("s")
    g = c * NS + s
    has_extra = s < EXTRA

    pltpu.sync_copy(idxf_hbm.at[pl.ds(s * KB, KB)], idxf_v.at[pl.ds(0, KB)])
    pltpu.sync_copy(seg_hbm.at[pl.ds(s * KB, KB)], seg_v.at[pl.ds(0, KB)])

    @pl.when(has_extra)
    def _():
        pltpu.sync_copy(idxf_hbm.at[NS * KB + s], idxf_v.at[KB])
        pltpu.sync_copy(seg_hbm.at[NS * KB + s], seg_v.at[KB])

    def xform_body(i, _):
        for d in range(CH // 16):
            v = idxf_v[i, pl.ds(d * 16, 16)]
            idxf_v[i, pl.ds(d * 16, 16)] = v * 2 + c
        return 0

    lax.fori_loop(0, KMAX, xform_body, 0)

    zero16 = jnp.zeros((16,), jnp.float32)
    ones16 = jnp.ones((16,), jnp.float32)

    def zrows_body(i, _):
        for d in range(DH // 16):
            rows0[i, pl.ds(d * 16, 16)] = zero16
        return 0

    lax.fori_loop(0, CH, zrows_body, 0)

    def zcnt_body(i, _):
        cnt_v[pl.ds(i * 16, 16)] = zero16
        return 0

    lax.fori_loop(0, ACC_ROWS // 16, zcnt_body, 0)

    base = s * ZR
    for k in range(ZR // CH):
        pltpu.sync_copy(rows0, acc.at[pl.ds(base + k * CH, CH)])
    rem = ZR % CH
    if rem:
        off = base + (ZR // CH) * CH
        pltpu.sync_copy(rows0.at[pl.ds(0, rem)], acc.at[pl.ds(off, rem)])

    plsc.subcore_barrier()

    def fire(j, buf, sem):
        pltpu.async_copy(feat_hbm.at[idxf_v.at[j]], buf, sem)

    def drain(j, buf, sem):
        pltpu.make_async_copy(feat_hbm.at[idxf_v.at[j]], buf, sem).wait()

    def fire_s(j, buf, sem):
        pltpu.async_copy(buf, acc.at[seg_v.at[j]], sem, add=True)

    def wait_s(j, buf, sem):
        pltpu.make_async_copy(buf, acc.at[seg_v.at[j]], sem).wait()

    def count(j):
        @pl.when(lax.rem(j, 2) == c)
        def _():
            for l in range(CH // 16):
                sv = seg_v[j, pl.ds(l * 16, 16)]
                plsc.addupdate_scatter(cnt_v, [sv], ones16)

    bufs = (rows0, rows1, rows2, rows3)
    sems = (gs0, gs1, gs2, gs3)
    ssems = (ss0, ss1, ss2, ss3)
    NB = len(bufs)

    def slot(j, b, do_wait=True, fire_next=True):
        drain(j, bufs[b], sems[b])
        fire_s(j, bufs[b], ssems[b])
        count(j)
        bp = (b - 1) % NB
        if do_wait:
            wait_s(j - 1, bufs[bp], ssems[bp])
        if fire_next:
            fire(j + NB - 1, bufs[bp], sems[bp])

    for b in range(NB - 1):
        fire(b, bufs[b], sems[b])
    slot(0, 0, do_wait=False)
    slot(1, 1)
    slot(2, 2)
    slot(3, 3)

    NQ_LO, NQ_HI = 1, (KB - 2 * NB + 1) // NB

    def quad_body(q, _):
        j0 = NB * q
        for b in range(NB):
            slot(j0 + b, b)
        return 0

    lax.fori_loop(NQ_LO, NQ_HI + 1, quad_body, 0)
    for j in range(NB * (NQ_HI + 1), KB):
        slot(j, j % NB, fire_next=(j + NB - 1 < KB))
    wait_s(KB - 1, bufs[(KB - 1) % NB], ssems[(KB - 1) % NB])

    @pl.when(has_extra)
    def _():
        fire(KB, rows0, gs0)
        drain(KB, rows0, gs0)
        fire_s(KB, rows0, ss0)
        count(KB)
        wait_s(KB, rows0, ss0)

    plsc.subcore_barrier()

    @pl.when(c == 0)
    def _():
        pltpu.sync_copy(acc.at[pl.ds(base, ZR)],
                        part_hbm.at[pl.ds(base, ZR), pl.ds(0, DH)])

    @pl.when(c == 1)
    def _():
        pltpu.sync_copy(acc.at[pl.ds(base, ZR)],
                        part_hbm.at[pl.ds(base, ZR), pl.ds(DH, DH)])

    pltpu.sync_copy(cnt_v, cnt_hbm.at[g])


_sc_call = pl.kernel(
    _sc_body,
    out_type=[
        jax.ShapeDtypeStruct((ACC_ROWS, D), jnp.float32),
        jax.ShapeDtypeStruct((NW, ACC_ROWS), jnp.float32),
    ],
    mesh=plsc.VectorSubcoreMesh(
        core_axis_name="c", subcore_axis_name="s",
        num_cores=NC, num_subcores=NS),
    compiler_params=pltpu.CompilerParams(
        use_tc_tiling_on_sc=False, needs_layout_passes=False),
    scratch_types=[
        pltpu.VMEM((KMAX, CH), jnp.int32),
        pltpu.VMEM((KMAX, CH), jnp.int32),
        pltpu.VMEM((CH, DH), jnp.float32),
        pltpu.VMEM((CH, DH), jnp.float32),
        pltpu.VMEM((CH, DH), jnp.float32),
        pltpu.VMEM((CH, DH), jnp.float32),
        pltpu.VMEM((ACC_ROWS,), jnp.float32),
        pltpu.VMEM_SHARED((ACC_ROWS, DH), jnp.float32),
        pltpu.SemaphoreType.DMA,
        pltpu.SemaphoreType.DMA,
        pltpu.SemaphoreType.DMA,
        pltpu.SemaphoreType.DMA,
        pltpu.SemaphoreType.DMA,
        pltpu.SemaphoreType.DMA,
        pltpu.SemaphoreType.DMA,
        pltpu.SemaphoreType.DMA,
    ],
)


def _comb_body(p_ref, c_ref, o_ref):
    counts = jnp.maximum(jnp.sum(c_ref[...], axis=1, keepdims=True), 1.0)
    o_ref[...] = p_ref[...] / counts


_COMB_BS = 2000


def _combine(part, cnt_t):
    return pl.pallas_call(
        _comb_body,
        grid=(N_CUR // _COMB_BS,),
        in_specs=[
            pl.BlockSpec((_COMB_BS, D), lambda i: (i, 0)),
            pl.BlockSpec((_COMB_BS, NW), lambda i: (i, 0)),
        ],
        out_specs=pl.BlockSpec((_COMB_BS, D), lambda i: (i, 0)),
        out_shape=jax.ShapeDtypeStruct((N_CUR, D), jnp.float32),
    )(part, cnt_t)


@jax.jit
def kernel(features, coors_inv_last, coors_inv, coors):
    del coors
    feat_h = features.reshape(-1, DH)
    idxf = coors_inv_last.astype(jnp.int32).reshape(NCHUNK, CH)
    seg = coors_inv.astype(jnp.int32).reshape(NCHUNK, CH)
    part, cnt = _sc_call(feat_h, idxf, seg)
    return _combine(part, cnt.T)
